# Initial kernel scaffold; baseline (speedup 1.0000x reference)
import jax
import jax.numpy as jnp
from jax.experimental import pallas as pl


def kernel(z, A, neighbor_indices, affinity_bins, P_w, y_w, W_w, c_bins, residual_weight):
    raise NotImplementedError("write your pallas kernel here")


# speedup vs baseline: 1.5522x; 1.5522x over previous
import functools

import jax
import jax.numpy as jnp
from jax import lax
from jax.experimental import pallas as pl
from jax.experimental.pallas import tpu as pltpu
from jax.experimental.pallas import tpu_sc as plsc

N_PAD = 10240
K = 32
D = 128
H = 8
HD = D // H
VT = 2 * D
NBP = 16

NW = 32
NODES_PER_W = N_PAD // NW
GCHUNK = 4
EDGES_PER_STEP = GCHUNK * K

BN1 = 256
BN3 = 64


def _proj_body(z_ref, w_ref, u_ref, vt_ref):
    b = jnp.dot(z_ref[...], w_ref[...], preferred_element_type=jnp.float32)
    u_ref[...] = b[:, :D]
    vt_ref[...] = b[:, D:]


def _proj(z_p, wcat):
    return pl.pallas_call(
        _proj_body,
        grid=(N_PAD // BN1,),
        in_specs=[
            pl.BlockSpec((BN1, D), lambda i: (i, 0)),
            pl.BlockSpec((D, 3 * D), lambda i: (0, 0)),
        ],
        out_specs=[
            pl.BlockSpec((BN1, D), lambda i: (i, 0)),
            pl.BlockSpec((BN1, VT), lambda i: (i, 0)),
        ],
        out_shape=[
            jax.ShapeDtypeStruct((N_PAD, D), jnp.float32),
            jax.ShapeDtypeStruct((N_PAD, VT), jnp.float32),
        ],
    )(z_p, wcat)


def _gather_body(idx_hbm, vt_hbm, out_hbm, idx_v, rows_v, sem):
    wid = lax.axis_index("s") * 2 + lax.axis_index("c")
    base_edge = wid * (NODES_PER_W * K)

    def step(ci, _):
        eb = base_edge + ci * EDGES_PER_STEP
        pltpu.sync_copy(idx_hbm.at[pl.ds(eb, EDGES_PER_STEP)], idx_v)
        pltpu.async_copy(vt_hbm.at[idx_v], rows_v, sem).wait()
        pltpu.sync_copy(rows_v, out_hbm.at[pl.ds(eb, EDGES_PER_STEP), :])
        return 0

    lax.fori_loop(0, NODES_PER_W // GCHUNK, step, 0)


@functools.cache
def _make_gather():
    return pl.kernel(
        _gather_body,
        mesh=plsc.VectorSubcoreMesh(core_axis_name="c", subcore_axis_name="s"),
        out_type=jax.ShapeDtypeStruct((N_PAD * K, VT), jnp.float32),
        scratch_types=[
            pltpu.VMEM((EDGES_PER_STEP,), jnp.int32),
            pltpu.VMEM((EDGES_PER_STEP, VT), jnp.float32),
            pltpu.SemaphoreType.DMA,
        ],
    )


def _gather(idx_flat, vt):
    return _make_gather()(idx_flat, vt)


def _dense_body(z_ref, u_ref, g_ref, bins_ref, y_ref, c_ref, rw_ref, out_ref):
    g = g_ref[...]
    gv3 = g[:, :D].reshape(BN3, K, D)
    u3 = u_ref[...][:, None, :]
    h = u3 + gv3
    h = jnp.maximum(h, 0.2 * h)
    h2 = h.reshape(BN3 * K, D)
    scores = lax.dot_general(
        h2, y_ref[...],
        dimension_numbers=(((1,), (1,)), ((), ())),
        preferred_element_type=jnp.float32,
    )
    bins = bins_ref[...]
    ib = lax.broadcasted_iota(jnp.int32, (BN3, K, NBP), 2)
    oh = (ib == bins[:, :, None]).astype(jnp.float32)
    cw = jnp.dot(oh.reshape(BN3 * K, NBP), c_ref[...],
                 preferred_element_type=jnp.float32)
    s = (scores + cw).reshape(BN3, K, H)
    m = jnp.max(s, axis=1, keepdims=True)
    e = jnp.exp(s - m)
    w = e / jnp.sum(e, axis=1, keepdims=True)

    gt3 = g[:, D:].reshape(BN3, K, D)
    outs = []
    for hh in range(H):
        wh = w[:, :, hh]
        gth = gt3[:, :, hh * HD:(hh + 1) * HD]
        outs.append(jnp.sum(wh[:, :, None] * gth, axis=1))
    agg = jnp.concatenate(outs, axis=-1)

    res = agg + rw_ref[0, 0] * z_ref[...]
    out_ref[...] = jnp.maximum(res, 0.2 * res)


def _dense(z_p, u, gathered, bins_p, y_w, c_pad, rw):
    return pl.pallas_call(
        _dense_body,
        grid=(N_PAD // BN3,),
        in_specs=[
            pl.BlockSpec((BN3, D), lambda i: (i, 0)),
            pl.BlockSpec((BN3, D), lambda i: (i, 0)),
            pl.BlockSpec((BN3 * K, VT), lambda i: (i, 0)),
            pl.BlockSpec((BN3, K), lambda i: (i, 0)),
            pl.BlockSpec((H, D), lambda i: (0, 0)),
            pl.BlockSpec((NBP, H), lambda i: (0, 0)),
            pl.BlockSpec((1, 1), lambda i: (0, 0), memory_space=pltpu.SMEM),
        ],
        out_specs=pl.BlockSpec((BN3, D), lambda i: (i, 0)),
        out_shape=jax.ShapeDtypeStruct((N_PAD, D), jnp.float32),
    )(z_p, u, gathered, bins_p, y_w, c_pad, rw)


def kernel(z, A, neighbor_indices, affinity_bins, P_w, y_w, W_w, c_bins,
           residual_weight):
    n, d = z.shape
    pad = N_PAD - n
    z_p = jnp.pad(z, ((0, pad), (0, 0)))
    ni_p = jnp.pad(neighbor_indices, ((0, pad), (0, 0)))
    ab_p = jnp.pad(affinity_bins, ((0, pad), (0, 0)))
    c_pad = jnp.pad(c_bins, ((0, NBP - c_bins.shape[0]), (0, 0)))
    wcat = jnp.concatenate(
        [P_w[:, :D].T, P_w[:, D:].T, W_w.T], axis=1)

    u, vt = _proj(z_p, wcat)
    gathered = _gather(ni_p.reshape(-1), vt)
    rw = residual_weight.reshape(1, 1)
    out_p = _dense(z_p, u, gathered, ab_p, y_w, c_pad, rw)
    return out_p[:n]


# speedup vs baseline: 1.7372x; 1.1192x over previous
import functools

import jax
import jax.numpy as jnp
from jax import lax
from jax.experimental import pallas as pl
from jax.experimental.pallas import tpu as pltpu
from jax.experimental.pallas import tpu_sc as plsc

N_PAD = 10240
K = 32
D = 128
H = 8
HD = D // H
VT = 2 * D
NBP = 16

NW = 32
NODES_PER_W = N_PAD // NW
GCHUNK = 4
EDGES_PER_STEP = GCHUNK * K

BN1 = 256
BN3 = 64


def _proj_body(z_ref, w_ref, u_ref, vt_ref):
    b = jnp.dot(z_ref[...], w_ref[...], preferred_element_type=jnp.float32)
    u_ref[...] = b[:, :D]
    vt_ref[...] = b[:, D:]


def _proj(z_p, wcat):
    return pl.pallas_call(
        _proj_body,
        grid=(N_PAD // BN1,),
        in_specs=[
            pl.BlockSpec((BN1, D), lambda i: (i, 0)),
            pl.BlockSpec((D, 3 * D), lambda i: (0, 0)),
        ],
        out_specs=[
            pl.BlockSpec((BN1, D), lambda i: (i, 0)),
            pl.BlockSpec((BN1, VT), lambda i: (i, 0)),
        ],
        out_shape=[
            jax.ShapeDtypeStruct((N_PAD, D), jnp.float32),
            jax.ShapeDtypeStruct((N_PAD, VT), jnp.float32),
        ],
    )(z_p, wcat)


def _gather_body(idx_hbm, vt_hbm, out_hbm, ib, rows0, rows1, g0, g1, w0, w1):
    EPS = EDGES_PER_STEP
    STEPS = NODES_PER_W // GCHUNK
    wid = lax.axis_index("s") * 2 + lax.axis_index("c")
    ebase = wid * (NODES_PER_W * K)

    pltpu.sync_copy(idx_hbm.at[pl.ds(ebase, NODES_PER_W * K)], ib)

    def ib_at(ci):
        return ib.at[pl.ds(ci * EPS, EPS)]

    def out_at(ci):
        return out_hbm.at[pl.ds(ebase + ci * EPS, EPS), :]

    pltpu.async_copy(vt_hbm.at[ib_at(0)], rows0, g0)
    pltpu.async_copy(vt_hbm.at[ib_at(1)], rows1, g1)
    pltpu.make_async_copy(vt_hbm.at[ib_at(0)], rows0, g0).wait()
    pltpu.async_copy(rows0, out_at(0), w0)
    pltpu.make_async_copy(vt_hbm.at[ib_at(1)], rows1, g1).wait()
    pltpu.async_copy(rows1, out_at(1), w1)

    def pair(j, _):
        i0 = 2 * j
        i1 = i0 + 1
        pltpu.make_async_copy(rows0, out_at(0), w0).wait()
        pltpu.async_copy(vt_hbm.at[ib_at(i0)], rows0, g0)
        pltpu.make_async_copy(rows1, out_at(0), w1).wait()
        pltpu.async_copy(vt_hbm.at[ib_at(i1)], rows1, g1)
        pltpu.make_async_copy(vt_hbm.at[ib_at(i0)], rows0, g0).wait()
        pltpu.async_copy(rows0, out_at(i0), w0)
        pltpu.make_async_copy(vt_hbm.at[ib_at(i1)], rows1, g1).wait()
        pltpu.async_copy(rows1, out_at(i1), w1)
        return 0

    lax.fori_loop(1, STEPS // 2, pair, 0)
    pltpu.make_async_copy(rows0, out_at(0), w0).wait()
    pltpu.make_async_copy(rows1, out_at(0), w1).wait()


@functools.cache
def _make_gather():
    return pl.kernel(
        _gather_body,
        mesh=plsc.VectorSubcoreMesh(core_axis_name="c", subcore_axis_name="s"),
        out_type=jax.ShapeDtypeStruct((N_PAD * K, VT), jnp.float32),
        scratch_types=[
            pltpu.VMEM((NODES_PER_W * K,), jnp.int32),
            pltpu.VMEM((EDGES_PER_STEP, VT), jnp.float32),
            pltpu.VMEM((EDGES_PER_STEP, VT), jnp.float32),
            pltpu.SemaphoreType.DMA,
            pltpu.SemaphoreType.DMA,
            pltpu.SemaphoreType.DMA,
            pltpu.SemaphoreType.DMA,
        ],
    )


def _gather(idx_flat, vt):
    return _make_gather()(idx_flat, vt)


def _dense_body(z_ref, u_ref, g_ref, bins_ref, y_ref, c_ref, rw_ref, out_ref):
    g = g_ref[...]
    gv3 = g[:, :D].reshape(BN3, K, D)
    u3 = u_ref[...][:, None, :]
    h = u3 + gv3
    h = jnp.maximum(h, 0.2 * h)
    h2 = h.reshape(BN3 * K, D)
    scores = lax.dot_general(
        h2, y_ref[...],
        dimension_numbers=(((1,), (1,)), ((), ())),
        preferred_element_type=jnp.float32,
    )
    bins = bins_ref[...]
    ib = lax.broadcasted_iota(jnp.int32, (BN3, K, NBP), 2)
    oh = (ib == bins[:, :, None]).astype(jnp.float32)
    cw = jnp.dot(oh.reshape(BN3 * K, NBP), c_ref[...],
                 preferred_element_type=jnp.float32)
    s = (scores + cw).reshape(BN3, K, H)
    m = jnp.max(s, axis=1, keepdims=True)
    e = jnp.exp(s - m)
    w = e / jnp.sum(e, axis=1, keepdims=True)

    gt3 = g[:, D:].reshape(BN3, K, D)
    outs = []
    for hh in range(H):
        wh = w[:, :, hh]
        gth = gt3[:, :, hh * HD:(hh + 1) * HD]
        outs.append(jnp.sum(wh[:, :, None] * gth, axis=1))
    agg = jnp.concatenate(outs, axis=-1)

    res = agg + rw_ref[0, 0] * z_ref[...]
    out_ref[...] = jnp.maximum(res, 0.2 * res)


def _dense(z_p, u, gathered, bins_p, y_w, c_pad, rw):
    return pl.pallas_call(
        _dense_body,
        grid=(N_PAD // BN3,),
        in_specs=[
            pl.BlockSpec((BN3, D), lambda i: (i, 0)),
            pl.BlockSpec((BN3, D), lambda i: (i, 0)),
            pl.BlockSpec((BN3 * K, VT), lambda i: (i, 0)),
            pl.BlockSpec((BN3, K), lambda i: (i, 0)),
            pl.BlockSpec((H, D), lambda i: (0, 0)),
            pl.BlockSpec((NBP, H), lambda i: (0, 0)),
            pl.BlockSpec((1, 1), lambda i: (0, 0), memory_space=pltpu.SMEM),
        ],
        out_specs=pl.BlockSpec((BN3, D), lambda i: (i, 0)),
        out_shape=jax.ShapeDtypeStruct((N_PAD, D), jnp.float32),
    )(z_p, u, gathered, bins_p, y_w, c_pad, rw)


def kernel(z, A, neighbor_indices, affinity_bins, P_w, y_w, W_w, c_bins,
           residual_weight):
    n, d = z.shape
    pad = N_PAD - n
    z_p = jnp.pad(z, ((0, pad), (0, 0)))
    ni_p = jnp.pad(neighbor_indices, ((0, pad), (0, 0)))
    ab_p = jnp.pad(affinity_bins, ((0, pad), (0, 0)))
    c_pad = jnp.pad(c_bins, ((0, NBP - c_bins.shape[0]), (0, 0)))
    wcat = jnp.concatenate(
        [P_w[:, :D].T, P_w[:, D:].T, W_w.T], axis=1)

    u, vt = _proj(z_p, wcat)
    gathered = _gather(ni_p.reshape(-1), vt)
    rw = residual_weight.reshape(1, 1)
    out_p = _dense(z_p, u, gathered, ab_p, y_w, c_pad, rw)
    return out_p[:n]


# speedup vs baseline: 1.8905x; 1.0882x over previous
import functools

import jax
import jax.numpy as jnp
from jax import lax
from jax.experimental import pallas as pl
from jax.experimental.pallas import tpu as pltpu
from jax.experimental.pallas import tpu_sc as plsc

N_PAD = 10240
K = 32
D = 128
H = 8
HD = D // H
VT = 2 * D
NBP = 16

NW = 32
NODES_PER_W = N_PAD // NW
GCHUNK = 4
EDGES_PER_STEP = GCHUNK * K

BN1 = 256
BN3 = 64


def _proj_body(z_ref, w_ref, u_ref, vt_ref):
    b = jnp.dot(z_ref[...], w_ref[...], preferred_element_type=jnp.float32)
    u_ref[...] = b[:, :D]
    vv = b[:, D:2 * D].astype(jnp.bfloat16)
    tt = b[:, 2 * D:].astype(jnp.bfloat16)
    lo = lax.bitcast_convert_type(vv, jnp.uint16).astype(jnp.uint32)
    hi = lax.bitcast_convert_type(tt, jnp.uint16).astype(jnp.uint32)
    vt_ref[...] = lax.bitcast_convert_type(lo | (hi << 16), jnp.int32)


def _proj(z_p, wcat):
    return pl.pallas_call(
        _proj_body,
        grid=(N_PAD // BN1,),
        in_specs=[
            pl.BlockSpec((BN1, D), lambda i: (i, 0)),
            pl.BlockSpec((D, 3 * D), lambda i: (0, 0)),
        ],
        out_specs=[
            pl.BlockSpec((BN1, D), lambda i: (i, 0)),
            pl.BlockSpec((BN1, D), lambda i: (i, 0)),
        ],
        out_shape=[
            jax.ShapeDtypeStruct((N_PAD, D), jnp.float32),
            jax.ShapeDtypeStruct((N_PAD, D), jnp.int32),
        ],
    )(z_p, wcat)


def _gather_body(idx_hbm, vt_hbm, out_hbm, ib, rows0, rows1, g0, g1, w0, w1):
    EPS = EDGES_PER_STEP
    STEPS = NODES_PER_W // GCHUNK
    wid = lax.axis_index("s") * 2 + lax.axis_index("c")
    ebase = wid * (NODES_PER_W * K)

    pltpu.sync_copy(idx_hbm.at[pl.ds(ebase, NODES_PER_W * K)], ib)

    def ib_at(ci):
        return ib.at[pl.ds(ci * EPS, EPS)]

    def out_at(ci):
        return out_hbm.at[pl.ds(ebase + ci * EPS, EPS), :]

    pltpu.async_copy(vt_hbm.at[ib_at(0)], rows0, g0)
    pltpu.async_copy(vt_hbm.at[ib_at(1)], rows1, g1)
    pltpu.make_async_copy(vt_hbm.at[ib_at(0)], rows0, g0).wait()
    pltpu.async_copy(rows0, out_at(0), w0)
    pltpu.make_async_copy(vt_hbm.at[ib_at(1)], rows1, g1).wait()
    pltpu.async_copy(rows1, out_at(1), w1)

    def pair(j, _):
        i0 = 2 * j
        i1 = i0 + 1
        pltpu.make_async_copy(rows0, out_at(0), w0).wait()
        pltpu.async_copy(vt_hbm.at[ib_at(i0)], rows0, g0)
        pltpu.make_async_copy(rows1, out_at(0), w1).wait()
        pltpu.async_copy(vt_hbm.at[ib_at(i1)], rows1, g1)
        pltpu.make_async_copy(vt_hbm.at[ib_at(i0)], rows0, g0).wait()
        pltpu.async_copy(rows0, out_at(i0), w0)
        pltpu.make_async_copy(vt_hbm.at[ib_at(i1)], rows1, g1).wait()
        pltpu.async_copy(rows1, out_at(i1), w1)
        return 0

    lax.fori_loop(1, STEPS // 2, pair, 0)
    pltpu.make_async_copy(rows0, out_at(0), w0).wait()
    pltpu.make_async_copy(rows1, out_at(0), w1).wait()


@functools.cache
def _make_gather():
    return pl.kernel(
        _gather_body,
        mesh=plsc.VectorSubcoreMesh(core_axis_name="c", subcore_axis_name="s"),
        out_type=jax.ShapeDtypeStruct((N_PAD * K, D), jnp.int32),
        scratch_types=[
            pltpu.VMEM((NODES_PER_W * K,), jnp.int32),
            pltpu.VMEM((EDGES_PER_STEP, D), jnp.int32),
            pltpu.VMEM((EDGES_PER_STEP, D), jnp.int32),
            pltpu.SemaphoreType.DMA,
            pltpu.SemaphoreType.DMA,
            pltpu.SemaphoreType.DMA,
            pltpu.SemaphoreType.DMA,
        ],
    )


def _gather(idx_flat, vt):
    return _make_gather()(idx_flat, vt)


def _dense_body(z_ref, u_ref, g_ref, bins_ref, y_ref, c_ref, rw_ref, out_ref):
    g = lax.bitcast_convert_type(g_ref[...], jnp.uint32)
    vv = lax.bitcast_convert_type(
        (g & 0xFFFF).astype(jnp.uint16), jnp.bfloat16).astype(jnp.float32)
    tt = lax.bitcast_convert_type(
        (g >> 16).astype(jnp.uint16), jnp.bfloat16).astype(jnp.float32)
    gv3 = vv.reshape(BN3, K, D)
    u3 = u_ref[...][:, None, :]
    h = u3 + gv3
    h = jnp.maximum(h, 0.2 * h)
    h2 = h.reshape(BN3 * K, D)
    scores = lax.dot_general(
        h2, y_ref[...],
        dimension_numbers=(((1,), (1,)), ((), ())),
        preferred_element_type=jnp.float32,
    )
    bins = bins_ref[...]
    ib = lax.broadcasted_iota(jnp.int32, (BN3, K, NBP), 2)
    oh = (ib == bins[:, :, None]).astype(jnp.float32)
    cw = jnp.dot(oh.reshape(BN3 * K, NBP), c_ref[...],
                 preferred_element_type=jnp.float32)
    s = (scores + cw).reshape(BN3, K, H)
    m = jnp.max(s, axis=1, keepdims=True)
    e = jnp.exp(s - m)
    w = e / jnp.sum(e, axis=1, keepdims=True)

    gt3 = tt.reshape(BN3, K, D)
    outs = []
    for hh in range(H):
        wh = w[:, :, hh]
        gth = gt3[:, :, hh * HD:(hh + 1) * HD]
        outs.append(jnp.sum(wh[:, :, None] * gth, axis=1))
    agg = jnp.concatenate(outs, axis=-1)

    res = agg + rw_ref[0, 0] * z_ref[...]
    out_ref[...] = jnp.maximum(res, 0.2 * res)


def _dense(z_p, u, gathered, bins_p, y_w, c_pad, rw):
    return pl.pallas_call(
        _dense_body,
        grid=(N_PAD // BN3,),
        in_specs=[
            pl.BlockSpec((BN3, D), lambda i: (i, 0)),
            pl.BlockSpec((BN3, D), lambda i: (i, 0)),
            pl.BlockSpec((BN3 * K, D), lambda i: (i, 0)),
            pl.BlockSpec((BN3, K), lambda i: (i, 0)),
            pl.BlockSpec((H, D), lambda i: (0, 0)),
            pl.BlockSpec((NBP, H), lambda i: (0, 0)),
            pl.BlockSpec((1, 1), lambda i: (0, 0), memory_space=pltpu.SMEM),
        ],
        out_specs=pl.BlockSpec((BN3, D), lambda i: (i, 0)),
        out_shape=jax.ShapeDtypeStruct((N_PAD, D), jnp.float32),
    )(z_p, u, gathered, bins_p, y_w, c_pad, rw)


def kernel(z, A, neighbor_indices, affinity_bins, P_w, y_w, W_w, c_bins,
           residual_weight):
    n, d = z.shape
    pad = N_PAD - n
    z_p = jnp.pad(z, ((0, pad), (0, 0)))
    ni_p = jnp.pad(neighbor_indices, ((0, pad), (0, 0)))
    ab_p = jnp.pad(affinity_bins, ((0, pad), (0, 0)))
    c_pad = jnp.pad(c_bins, ((0, NBP - c_bins.shape[0]), (0, 0)))
    wcat = jnp.concatenate(
        [P_w[:, :D].T, P_w[:, D:].T, W_w.T], axis=1)

    u, vt = _proj(z_p, wcat)
    gathered = _gather(ni_p.reshape(-1), vt)
    rw = residual_weight.reshape(1, 1)
    out_p = _dense(z_p, u, gathered, ab_p, y_w, c_pad, rw)
    return out_p[:n]


# speedup vs baseline: 1.9245x; 1.0180x over previous
import functools

import jax
import jax.numpy as jnp
from jax import lax
from jax.experimental import pallas as pl
from jax.experimental.pallas import tpu as pltpu
from jax.experimental.pallas import tpu_sc as plsc

N_PAD = 10240
K = 32
D = 128
H = 8
HD = D // H
VT = 2 * D
NBP = 16

NW = 32
NODES_PER_W = N_PAD // NW
GCHUNK = 8
EDGES_PER_STEP = GCHUNK * K

BN1 = 256
BN3 = 64


def _proj_body(z_ref, w_ref, u_ref, vt_ref):
    b = jnp.dot(z_ref[...], w_ref[...], preferred_element_type=jnp.float32)
    u_ref[...] = b[:, :D]
    vv = b[:, D:2 * D].astype(jnp.bfloat16)
    tt = b[:, 2 * D:].astype(jnp.bfloat16)
    lo = lax.bitcast_convert_type(vv, jnp.uint16).astype(jnp.uint32)
    hi = lax.bitcast_convert_type(tt, jnp.uint16).astype(jnp.uint32)
    vt_ref[...] = lax.bitcast_convert_type(lo | (hi << 16), jnp.int32)


def _proj(z_p, wcat):
    return pl.pallas_call(
        _proj_body,
        grid=(N_PAD // BN1,),
        in_specs=[
            pl.BlockSpec((BN1, D), lambda i: (i, 0)),
            pl.BlockSpec((D, 3 * D), lambda i: (0, 0)),
        ],
        out_specs=[
            pl.BlockSpec((BN1, D), lambda i: (i, 0)),
            pl.BlockSpec((BN1, D), lambda i: (i, 0)),
        ],
        out_shape=[
            jax.ShapeDtypeStruct((N_PAD, D), jnp.float32),
            jax.ShapeDtypeStruct((N_PAD, D), jnp.int32),
        ],
    )(z_p, wcat)


def _gather_body(idx_hbm, vt_hbm, out_hbm, ib, rows0, rows1, g0, g1, w0, w1):
    EPS = EDGES_PER_STEP
    STEPS = NODES_PER_W // GCHUNK
    wid = lax.axis_index("s") * 2 + lax.axis_index("c")
    ebase = wid * (NODES_PER_W * K)

    pltpu.sync_copy(idx_hbm.at[pl.ds(ebase, NODES_PER_W * K)], ib)

    def ib_at(ci):
        return ib.at[pl.ds(ci * EPS, EPS)]

    def out_at(ci):
        return out_hbm.at[pl.ds(ebase + ci * EPS, EPS), :]

    pltpu.async_copy(vt_hbm.at[ib_at(0)], rows0, g0)
    pltpu.async_copy(vt_hbm.at[ib_at(1)], rows1, g1)
    pltpu.make_async_copy(vt_hbm.at[ib_at(0)], rows0, g0).wait()
    pltpu.async_copy(rows0, out_at(0), w0)
    pltpu.make_async_copy(vt_hbm.at[ib_at(1)], rows1, g1).wait()
    pltpu.async_copy(rows1, out_at(1), w1)

    def pair(j, _):
        i0 = 2 * j
        i1 = i0 + 1
        pltpu.make_async_copy(rows0, out_at(0), w0).wait()
        pltpu.async_copy(vt_hbm.at[ib_at(i0)], rows0, g0)
        pltpu.make_async_copy(rows1, out_at(0), w1).wait()
        pltpu.async_copy(vt_hbm.at[ib_at(i1)], rows1, g1)
        pltpu.make_async_copy(vt_hbm.at[ib_at(i0)], rows0, g0).wait()
        pltpu.async_copy(rows0, out_at(i0), w0)
        pltpu.make_async_copy(vt_hbm.at[ib_at(i1)], rows1, g1).wait()
        pltpu.async_copy(rows1, out_at(i1), w1)
        return 0

    lax.fori_loop(1, STEPS // 2, pair, 0)
    pltpu.make_async_copy(rows0, out_at(0), w0).wait()
    pltpu.make_async_copy(rows1, out_at(0), w1).wait()


@functools.cache
def _make_gather():
    return pl.kernel(
        _gather_body,
        mesh=plsc.VectorSubcoreMesh(core_axis_name="c", subcore_axis_name="s"),
        out_type=jax.ShapeDtypeStruct((N_PAD * K, D), jnp.int32),
        scratch_types=[
            pltpu.VMEM((NODES_PER_W * K,), jnp.int32),
            pltpu.VMEM((EDGES_PER_STEP, D), jnp.int32),
            pltpu.VMEM((EDGES_PER_STEP, D), jnp.int32),
            pltpu.SemaphoreType.DMA,
            pltpu.SemaphoreType.DMA,
            pltpu.SemaphoreType.DMA,
            pltpu.SemaphoreType.DMA,
        ],
    )


def _gather(idx_flat, vt):
    return _make_gather()(idx_flat, vt)


def _dense_body(z_ref, u_ref, g_ref, bins_ref, y_ref, c_ref, rw_ref, out_ref):
    g = lax.bitcast_convert_type(g_ref[...], jnp.uint32)
    vv = lax.bitcast_convert_type(
        (g & 0xFFFF).astype(jnp.uint16), jnp.bfloat16).astype(jnp.float32)
    tt = lax.bitcast_convert_type(
        (g >> 16).astype(jnp.uint16), jnp.bfloat16).astype(jnp.float32)
    gv3 = vv.reshape(BN3, K, D)
    u3 = u_ref[...][:, None, :]
    h = u3 + gv3
    h = jnp.maximum(h, 0.2 * h)
    h2 = h.reshape(BN3 * K, D)
    scores = lax.dot_general(
        h2, y_ref[...],
        dimension_numbers=(((1,), (1,)), ((), ())),
        preferred_element_type=jnp.float32,
    )
    bins = bins_ref[...]
    ib = lax.broadcasted_iota(jnp.int32, (BN3, K, NBP), 2)
    oh = (ib == bins[:, :, None]).astype(jnp.float32)
    cw = jnp.dot(oh.reshape(BN3 * K, NBP), c_ref[...],
                 preferred_element_type=jnp.float32)
    s = (scores + cw).reshape(BN3, K, H)
    m = jnp.max(s, axis=1, keepdims=True)
    e = jnp.exp(s - m)
    w = e / jnp.sum(e, axis=1, keepdims=True)

    gt3 = tt.reshape(BN3, K, D)
    outs = []
    for hh in range(H):
        wh = w[:, :, hh]
        gth = gt3[:, :, hh * HD:(hh + 1) * HD]
        outs.append(jnp.sum(wh[:, :, None] * gth, axis=1))
    agg = jnp.concatenate(outs, axis=-1)

    res = agg + rw_ref[0, 0] * z_ref[...]
    out_ref[...] = jnp.maximum(res, 0.2 * res)


def _dense(z_p, u, gathered, bins_p, y_w, c_pad, rw):
    return pl.pallas_call(
        _dense_body,
        grid=(N_PAD // BN3,),
        in_specs=[
            pl.BlockSpec((BN3, D), lambda i: (i, 0)),
            pl.BlockSpec((BN3, D), lambda i: (i, 0)),
            pl.BlockSpec((BN3 * K, D), lambda i: (i, 0)),
            pl.BlockSpec((BN3, K), lambda i: (i, 0)),
            pl.BlockSpec((H, D), lambda i: (0, 0)),
            pl.BlockSpec((NBP, H), lambda i: (0, 0)),
            pl.BlockSpec((1, 1), lambda i: (0, 0), memory_space=pltpu.SMEM),
        ],
        out_specs=pl.BlockSpec((BN3, D), lambda i: (i, 0)),
        out_shape=jax.ShapeDtypeStruct((N_PAD, D), jnp.float32),
    )(z_p, u, gathered, bins_p, y_w, c_pad, rw)


def kernel(z, A, neighbor_indices, affinity_bins, P_w, y_w, W_w, c_bins,
           residual_weight):
    n, d = z.shape
    pad = N_PAD - n
    z_p = jnp.pad(z, ((0, pad), (0, 0)))
    ni_p = jnp.pad(neighbor_indices, ((0, pad), (0, 0)))
    ab_p = jnp.pad(affinity_bins, ((0, pad), (0, 0)))
    c_pad = jnp.pad(c_bins, ((0, NBP - c_bins.shape[0]), (0, 0)))
    wcat = jnp.concatenate(
        [P_w[:, :D].T, P_w[:, D:].T, W_w.T], axis=1)

    u, vt = _proj(z_p, wcat)
    gathered = _gather(ni_p.reshape(-1), vt)
    rw = residual_weight.reshape(1, 1)
    out_p = _dense(z_p, u, gathered, ab_p, y_w, c_pad, rw)
    return out_p[:n]


# speedup vs baseline: 1.9307x; 1.0032x over previous
import functools

import jax
import jax.numpy as jnp
from jax import lax
from jax.experimental import pallas as pl
from jax.experimental.pallas import tpu as pltpu
from jax.experimental.pallas import tpu_sc as plsc

N_PAD = 10240
K = 32
D = 128
H = 8
HD = D // H
VT = 2 * D
NBP = 16

NW = 32
NODES_PER_W = N_PAD // NW
GCHUNK = 4
EDGES_PER_STEP = GCHUNK * K

BN1 = 256
BN3 = 64


def _proj_body(z_ref, w_ref, u_ref, vt_ref):
    b = jnp.dot(z_ref[...], w_ref[...], preferred_element_type=jnp.float32)
    u_ref[...] = b[:, :D]
    vv = b[:, D:2 * D].astype(jnp.bfloat16)
    tt = b[:, 2 * D:].astype(jnp.bfloat16)
    lo = lax.bitcast_convert_type(vv, jnp.uint16).astype(jnp.uint32)
    hi = lax.bitcast_convert_type(tt, jnp.uint16).astype(jnp.uint32)
    vt_ref[...] = lax.bitcast_convert_type(lo | (hi << 16), jnp.int32)


def _proj(z_p, wcat):
    return pl.pallas_call(
        _proj_body,
        grid=(N_PAD // BN1,),
        in_specs=[
            pl.BlockSpec((BN1, D), lambda i: (i, 0)),
            pl.BlockSpec((D, 3 * D), lambda i: (0, 0)),
        ],
        out_specs=[
            pl.BlockSpec((BN1, D), lambda i: (i, 0)),
            pl.BlockSpec((BN1, D), lambda i: (i, 0)),
        ],
        out_shape=[
            jax.ShapeDtypeStruct((N_PAD, D), jnp.float32),
            jax.ShapeDtypeStruct((N_PAD, D), jnp.int32),
        ],
    )(z_p, wcat)


NBUF = 4


def _gather_body(idx_hbm, vt_hbm, out_hbm, ib, rows0, rows1, rows2, rows3,
                 g0, g1, g2, g3, w0, w1, w2, w3):
    EPS = EDGES_PER_STEP
    STEPS = NODES_PER_W // GCHUNK
    wid = lax.axis_index("s") * 2 + lax.axis_index("c")
    ebase = wid * (NODES_PER_W * K)
    bufs = (rows0, rows1, rows2, rows3)
    gsem = (g0, g1, g2, g3)
    wsem = (w0, w1, w2, w3)

    pltpu.sync_copy(idx_hbm.at[pl.ds(ebase, NODES_PER_W * K)], ib)

    def ib_at(ci):
        return ib.at[pl.ds(ci * EPS, EPS)]

    def out_at(ci):
        return out_hbm.at[pl.ds(ebase + ci * EPS, EPS), :]

    for b in range(NBUF):
        pltpu.async_copy(vt_hbm.at[ib_at(b)], bufs[b], gsem[b])
    for b in range(NBUF):
        pltpu.make_async_copy(vt_hbm.at[ib_at(b)], bufs[b], gsem[b]).wait()
        pltpu.async_copy(bufs[b], out_at(b), wsem[b])

    def rot(j, _):
        i0 = NBUF * j
        for b in range(NBUF):
            pltpu.make_async_copy(bufs[b], out_at(0), wsem[b]).wait()
            pltpu.async_copy(vt_hbm.at[ib_at(i0 + b)], bufs[b], gsem[b])
        for b in range(NBUF):
            pltpu.make_async_copy(vt_hbm.at[ib_at(i0 + b)], bufs[b],
                                  gsem[b]).wait()
            pltpu.async_copy(bufs[b], out_at(i0 + b), wsem[b])
        return 0

    lax.fori_loop(1, STEPS // NBUF, rot, 0)
    for b in range(NBUF):
        pltpu.make_async_copy(bufs[b], out_at(0), wsem[b]).wait()


@functools.cache
def _make_gather():
    return pl.kernel(
        _gather_body,
        mesh=plsc.VectorSubcoreMesh(core_axis_name="c", subcore_axis_name="s"),
        out_type=jax.ShapeDtypeStruct((N_PAD * K, D), jnp.int32),
        scratch_types=(
            [pltpu.VMEM((NODES_PER_W * K,), jnp.int32)]
            + [pltpu.VMEM((EDGES_PER_STEP, D), jnp.int32)] * NBUF
            + [pltpu.SemaphoreType.DMA] * (2 * NBUF)
        ),
    )


def _gather(idx_flat, vt):
    return _make_gather()(idx_flat, vt)


def _dense_body(z_ref, u_ref, g_ref, bins_ref, y_ref, c_ref, rw_ref, out_ref):
    g = lax.bitcast_convert_type(g_ref[...], jnp.uint32)
    vv = lax.bitcast_convert_type(
        (g & 0xFFFF).astype(jnp.uint16), jnp.bfloat16).astype(jnp.float32)
    tt = lax.bitcast_convert_type(
        (g >> 16).astype(jnp.uint16), jnp.bfloat16).astype(jnp.float32)
    gv3 = vv.reshape(BN3, K, D)
    u3 = u_ref[...][:, None, :]
    h = u3 + gv3
    h = jnp.maximum(h, 0.2 * h)
    h2 = h.reshape(BN3 * K, D)
    scores = lax.dot_general(
        h2, y_ref[...],
        dimension_numbers=(((1,), (1,)), ((), ())),
        preferred_element_type=jnp.float32,
    )
    bins = bins_ref[...]
    ib = lax.broadcasted_iota(jnp.int32, (BN3, K, NBP), 2)
    oh = (ib == bins[:, :, None]).astype(jnp.float32)
    cw = jnp.dot(oh.reshape(BN3 * K, NBP), c_ref[...],
                 preferred_element_type=jnp.float32)
    s = (scores + cw).reshape(BN3, K, H)
    m = jnp.max(s, axis=1, keepdims=True)
    e = jnp.exp(s - m)
    w = e / jnp.sum(e, axis=1, keepdims=True)

    gt3 = tt.reshape(BN3, K, D)
    outs = []
    for hh in range(H):
        wh = w[:, :, hh]
        gth = gt3[:, :, hh * HD:(hh + 1) * HD]
        outs.append(jnp.sum(wh[:, :, None] * gth, axis=1))
    agg = jnp.concatenate(outs, axis=-1)

    res = agg + rw_ref[0, 0] * z_ref[...]
    out_ref[...] = jnp.maximum(res, 0.2 * res)


def _dense(z_p, u, gathered, bins_p, y_w, c_pad, rw):
    return pl.pallas_call(
        _dense_body,
        grid=(N_PAD // BN3,),
        in_specs=[
            pl.BlockSpec((BN3, D), lambda i: (i, 0)),
            pl.BlockSpec((BN3, D), lambda i: (i, 0)),
            pl.BlockSpec((BN3 * K, D), lambda i: (i, 0)),
            pl.BlockSpec((BN3, K), lambda i: (i, 0)),
            pl.BlockSpec((H, D), lambda i: (0, 0)),
            pl.BlockSpec((NBP, H), lambda i: (0, 0)),
            pl.BlockSpec((1, 1), lambda i: (0, 0), memory_space=pltpu.SMEM),
        ],
        out_specs=pl.BlockSpec((BN3, D), lambda i: (i, 0)),
        out_shape=jax.ShapeDtypeStruct((N_PAD, D), jnp.float32),
    )(z_p, u, gathered, bins_p, y_w, c_pad, rw)


def kernel(z, A, neighbor_indices, affinity_bins, P_w, y_w, W_w, c_bins,
           residual_weight):
    n, d = z.shape
    pad = N_PAD - n
    z_p = jnp.pad(z, ((0, pad), (0, 0)))
    ni_p = jnp.pad(neighbor_indices, ((0, pad), (0, 0)))
    ab_p = jnp.pad(affinity_bins, ((0, pad), (0, 0)))
    c_pad = jnp.pad(c_bins, ((0, NBP - c_bins.shape[0]), (0, 0)))
    wcat = jnp.concatenate(
        [P_w[:, :D].T, P_w[:, D:].T, W_w.T], axis=1)

    u, vt = _proj(z_p, wcat)
    gathered = _gather(ni_p.reshape(-1), vt)
    rw = residual_weight.reshape(1, 1)
    out_p = _dense(z_p, u, gathered, ab_p, y_w, c_pad, rw)
    return out_p[:n]


# speedup vs baseline: 1.9856x; 1.0284x over previous
import functools

import jax
import jax.numpy as jnp
from jax import lax
from jax.experimental import pallas as pl
from jax.experimental.pallas import tpu as pltpu
from jax.experimental.pallas import tpu_sc as plsc

N_PAD = 10240
K = 32
D = 128
H = 8
HD = D // H
VT = 2 * D
NBP = 16

NW = 32
NODES_PER_W = N_PAD // NW
GCHUNK = 4
EDGES_PER_STEP = GCHUNK * K

BN1 = 256
BN3 = 64


def _proj_body(z_ref, w_ref, u_ref, vt_ref):
    b = jnp.dot(z_ref[...], w_ref[...], preferred_element_type=jnp.float32)
    u_ref[...] = b[:, :D]
    vv = b[:, D:2 * D].astype(jnp.bfloat16)
    tt = b[:, 2 * D:].astype(jnp.bfloat16)
    lo = lax.bitcast_convert_type(vv, jnp.uint16).astype(jnp.uint32)
    hi = lax.bitcast_convert_type(tt, jnp.uint16).astype(jnp.uint32)
    vt_ref[...] = lax.bitcast_convert_type(lo | (hi << 16), jnp.int32)


def _proj(z_p, wcat):
    return pl.pallas_call(
        _proj_body,
        grid=(N_PAD // BN1,),
        in_specs=[
            pl.BlockSpec((BN1, D), lambda i: (i, 0)),
            pl.BlockSpec((D, 3 * D), lambda i: (0, 0)),
        ],
        out_specs=[
            pl.BlockSpec((BN1, D), lambda i: (i, 0)),
            pl.BlockSpec((BN1, D), lambda i: (i, 0)),
        ],
        out_shape=[
            jax.ShapeDtypeStruct((N_PAD, D), jnp.float32),
            jax.ShapeDtypeStruct((N_PAD, D), jnp.int32),
        ],
    )(z_p, wcat)


NBUF = 4


def _gather_body(idx_hbm, vt_hbm, out_hbm, ib, rows0, rows1, rows2, rows3,
                 g0, g1, g2, g3, w0, w1, w2, w3, *, nodes_per_w):
    EPS = EDGES_PER_STEP
    STEPS = nodes_per_w // GCHUNK
    wid = lax.axis_index("s") * 2 + lax.axis_index("c")
    ebase = wid * (nodes_per_w * K)
    bufs = (rows0, rows1, rows2, rows3)
    gsem = (g0, g1, g2, g3)
    wsem = (w0, w1, w2, w3)

    pltpu.sync_copy(idx_hbm.at[pl.ds(ebase, nodes_per_w * K)], ib)

    def ib_at(ci):
        return ib.at[pl.ds(ci * EPS, EPS)]

    def out_at(ci):
        return out_hbm.at[pl.ds(ebase + ci * EPS, EPS), :]

    for b in range(NBUF):
        pltpu.async_copy(vt_hbm.at[ib_at(b)], bufs[b], gsem[b])
    for b in range(NBUF):
        pltpu.make_async_copy(vt_hbm.at[ib_at(b)], bufs[b], gsem[b]).wait()
        pltpu.async_copy(bufs[b], out_at(b), wsem[b])

    def rot(j, _):
        i0 = NBUF * j
        for b in range(NBUF):
            pltpu.make_async_copy(bufs[b], out_at(0), wsem[b]).wait()
            pltpu.async_copy(vt_hbm.at[ib_at(i0 + b)], bufs[b], gsem[b])
        for b in range(NBUF):
            pltpu.make_async_copy(vt_hbm.at[ib_at(i0 + b)], bufs[b],
                                  gsem[b]).wait()
            pltpu.async_copy(bufs[b], out_at(i0 + b), wsem[b])
        return 0

    lax.fori_loop(1, STEPS // NBUF, rot, 0)
    for b in range(NBUF):
        pltpu.make_async_copy(bufs[b], out_at(0), wsem[b]).wait()


@functools.cache
def _make_gather(nodes):
    nodes_per_w = nodes // NW
    return pl.kernel(
        functools.partial(_gather_body, nodes_per_w=nodes_per_w),
        mesh=plsc.VectorSubcoreMesh(core_axis_name="c", subcore_axis_name="s"),
        out_type=jax.ShapeDtypeStruct((nodes * K, D), jnp.int32),
        scratch_types=(
            [pltpu.VMEM((nodes_per_w * K,), jnp.int32)]
            + [pltpu.VMEM((EDGES_PER_STEP, D), jnp.int32)] * NBUF
            + [pltpu.SemaphoreType.DMA] * (2 * NBUF)
        ),
    )


def _gather(idx_flat, vt):
    return _make_gather(idx_flat.shape[0] // K)(idx_flat, vt)


def _dense_body(z_ref, u_ref, g_ref, bins_ref, y_ref, c_ref, rw_ref, out_ref):
    g = lax.bitcast_convert_type(g_ref[...], jnp.uint32)
    vv = lax.bitcast_convert_type(
        (g & 0xFFFF).astype(jnp.uint16), jnp.bfloat16).astype(jnp.float32)
    tt = lax.bitcast_convert_type(
        (g >> 16).astype(jnp.uint16), jnp.bfloat16).astype(jnp.float32)
    gv3 = vv.reshape(BN3, K, D)
    u3 = u_ref[...][:, None, :]
    h = u3 + gv3
    h = jnp.maximum(h, 0.2 * h)
    h2 = h.reshape(BN3 * K, D)
    scores = lax.dot_general(
        h2, y_ref[...],
        dimension_numbers=(((1,), (1,)), ((), ())),
        preferred_element_type=jnp.float32,
    )
    bins = bins_ref[...]
    ib = lax.broadcasted_iota(jnp.int32, (BN3, K, NBP), 2)
    oh = (ib == bins[:, :, None]).astype(jnp.float32)
    cw = jnp.dot(oh.reshape(BN3 * K, NBP), c_ref[...],
                 preferred_element_type=jnp.float32)
    s = (scores + cw).reshape(BN3, K, H)
    m = jnp.max(s, axis=1, keepdims=True)
    e = jnp.exp(s - m)
    w = e / jnp.sum(e, axis=1, keepdims=True)

    gt3 = tt.reshape(BN3, K, D)
    outs = []
    for hh in range(H):
        wh = w[:, :, hh]
        gth = gt3[:, :, hh * HD:(hh + 1) * HD]
        outs.append(jnp.sum(wh[:, :, None] * gth, axis=1))
    agg = jnp.concatenate(outs, axis=-1)

    res = agg + rw_ref[0, 0] * z_ref[...]
    out_ref[...] = jnp.maximum(res, 0.2 * res)


def _dense(z_p, u, gathered, bins_p, y_w, c_pad, rw):
    nrows = z_p.shape[0]
    return pl.pallas_call(
        _dense_body,
        grid=(nrows // BN3,),
        in_specs=[
            pl.BlockSpec((BN3, D), lambda i: (i, 0)),
            pl.BlockSpec((BN3, D), lambda i: (i, 0)),
            pl.BlockSpec((BN3 * K, D), lambda i: (i, 0)),
            pl.BlockSpec((BN3, K), lambda i: (i, 0)),
            pl.BlockSpec((H, D), lambda i: (0, 0)),
            pl.BlockSpec((NBP, H), lambda i: (0, 0)),
            pl.BlockSpec((1, 1), lambda i: (0, 0), memory_space=pltpu.SMEM),
        ],
        out_specs=pl.BlockSpec((BN3, D), lambda i: (i, 0)),
        out_shape=jax.ShapeDtypeStruct((nrows, D), jnp.float32),
    )(z_p, u, gathered, bins_p, y_w, c_pad, rw)


def kernel(z, A, neighbor_indices, affinity_bins, P_w, y_w, W_w, c_bins,
           residual_weight):
    n, d = z.shape
    pad = N_PAD - n
    z_p = jnp.pad(z, ((0, pad), (0, 0)))
    ni_p = jnp.pad(neighbor_indices, ((0, pad), (0, 0)))
    ab_p = jnp.pad(affinity_bins, ((0, pad), (0, 0)))
    c_pad = jnp.pad(c_bins, ((0, NBP - c_bins.shape[0]), (0, 0)))
    wcat = jnp.concatenate(
        [P_w[:, :D].T, P_w[:, D:].T, W_w.T], axis=1)

    u, vt = _proj(z_p, wcat)
    rw = residual_weight.reshape(1, 1)
    S = 2
    NS = N_PAD // S
    idx_flat = ni_p.reshape(-1)
    gathered = [_gather(idx_flat[s * NS * K:(s + 1) * NS * K], vt)
                for s in range(S)]
    outs = [
        _dense(z_p[s * NS:(s + 1) * NS], u[s * NS:(s + 1) * NS], gathered[s],
               ab_p[s * NS:(s + 1) * NS], y_w, c_pad, rw)
        for s in range(S)
    ]
    out_p = jnp.concatenate(outs, axis=0)
    return out_p[:n]


# speedup vs baseline: 3.1906x; 1.6069x over previous
import functools

import jax
import jax.numpy as jnp
from jax import lax
from jax.experimental import pallas as pl
from jax.experimental.pallas import tpu as pltpu
from jax.experimental.pallas import tpu_sc as plsc

N_PAD = 10240
K = 32
D = 128
H = 8
HD = D // H
VT = 2 * D
NBP = 16

NW = 32
NODES_PER_W = N_PAD // NW
GCHUNK = 4
EDGES_PER_STEP = GCHUNK * K

BN1 = 256
BN3 = 64


def _proj_body(z_ref, w_ref, u_ref, vt_ref):
    b = jnp.dot(z_ref[...], w_ref[...], preferred_element_type=jnp.float32)
    u_ref[...] = b[:, :D]
    vv = b[:, D:2 * D].astype(jnp.bfloat16)
    tt = b[:, 2 * D:].astype(jnp.bfloat16)
    lo = lax.bitcast_convert_type(vv, jnp.uint16).astype(jnp.uint32)
    hi = lax.bitcast_convert_type(tt, jnp.uint16).astype(jnp.uint32)
    vt_ref[...] = lax.bitcast_convert_type(lo | (hi << 16), jnp.int32)


def _proj(z_p, wcat):
    return pl.pallas_call(
        _proj_body,
        grid=(N_PAD // BN1,),
        in_specs=[
            pl.BlockSpec((BN1, D), lambda i: (i, 0)),
            pl.BlockSpec((D, 3 * D), lambda i: (0, 0)),
        ],
        out_specs=[
            pl.BlockSpec((BN1, D), lambda i: (i, 0)),
            pl.BlockSpec((BN1, D), lambda i: (i, 0)),
        ],
        out_shape=[
            jax.ShapeDtypeStruct((N_PAD, D), jnp.float32),
            jax.ShapeDtypeStruct((N_PAD, D), jnp.int32),
        ],
    )(z_p, wcat)


NBUF = 2


def _gather_body(idx_hbm, vt_hbm, out_hbm, ib, vt_sh, rows0, rows1,
                 g0, g1, w0, w1, *, nodes_per_w):
    EPS = EDGES_PER_STEP
    STEPS = nodes_per_w // GCHUNK
    wid = lax.axis_index("s") * 2 + lax.axis_index("c")
    ebase = wid * (nodes_per_w * K)
    bufs = (rows0, rows1)
    gsem = (g0, g1)
    wsem = (w0, w1)

    sid = lax.axis_index("s")
    SROWS = N_PAD // 16
    pltpu.sync_copy(vt_hbm.at[pl.ds(sid * SROWS, SROWS), :],
                    vt_sh.at[pl.ds(sid * SROWS, SROWS), :])
    plsc.subcore_barrier()

    pltpu.sync_copy(idx_hbm.at[pl.ds(ebase, nodes_per_w * K)], ib)

    def ib_at(ci):
        return ib.at[pl.ds(ci * EPS, EPS)]

    def out_at(ci):
        return out_hbm.at[pl.ds(ebase + ci * EPS, EPS), :]

    for b in range(NBUF):
        pltpu.async_copy(vt_sh.at[ib_at(b)], bufs[b], gsem[b])
    for b in range(NBUF):
        pltpu.make_async_copy(vt_sh.at[ib_at(b)], bufs[b], gsem[b]).wait()
        pltpu.async_copy(bufs[b], out_at(b), wsem[b])

    def rot(j, _):
        i0 = NBUF * j
        for b in range(NBUF):
            pltpu.make_async_copy(bufs[b], out_at(0), wsem[b]).wait()
            pltpu.async_copy(vt_sh.at[ib_at(i0 + b)], bufs[b], gsem[b])
        for b in range(NBUF):
            pltpu.make_async_copy(vt_sh.at[ib_at(i0 + b)], bufs[b],
                                  gsem[b]).wait()
            pltpu.async_copy(bufs[b], out_at(i0 + b), wsem[b])
        return 0

    lax.fori_loop(1, STEPS // NBUF, rot, 0)
    for b in range(NBUF):
        pltpu.make_async_copy(bufs[b], out_at(0), wsem[b]).wait()


@functools.cache
def _make_gather(nodes):
    nodes_per_w = nodes // NW
    return pl.kernel(
        functools.partial(_gather_body, nodes_per_w=nodes_per_w),
        mesh=plsc.VectorSubcoreMesh(core_axis_name="c", subcore_axis_name="s"),
        out_type=jax.ShapeDtypeStruct((nodes * K, D), jnp.int32),
        scratch_types=(
            [pltpu.VMEM((nodes_per_w * K,), jnp.int32),
             pltpu.VMEM_SHARED((N_PAD, D), jnp.int32)]
            + [pltpu.VMEM((EDGES_PER_STEP, D), jnp.int32)] * NBUF
            + [pltpu.SemaphoreType.DMA] * (2 * NBUF)
        ),
    )


def _gather(idx_flat, vt):
    return _make_gather(idx_flat.shape[0] // K)(idx_flat, vt)


def _dense_body(z_ref, u_ref, g_ref, bins_ref, y_ref, c_ref, rw_ref, out_ref):
    g = lax.bitcast_convert_type(g_ref[...], jnp.uint32)
    vv = lax.bitcast_convert_type(
        (g & 0xFFFF).astype(jnp.uint16), jnp.bfloat16).astype(jnp.float32)
    tt = lax.bitcast_convert_type(
        (g >> 16).astype(jnp.uint16), jnp.bfloat16).astype(jnp.float32)
    gv3 = vv.reshape(BN3, K, D)
    u3 = u_ref[...][:, None, :]
    h = u3 + gv3
    h = jnp.maximum(h, 0.2 * h)
    h2 = h.reshape(BN3 * K, D)
    scores = lax.dot_general(
        h2, y_ref[...],
        dimension_numbers=(((1,), (1,)), ((), ())),
        preferred_element_type=jnp.float32,
    )
    bins = bins_ref[...]
    ib = lax.broadcasted_iota(jnp.int32, (BN3, K, NBP), 2)
    oh = (ib == bins[:, :, None]).astype(jnp.float32)
    cw = jnp.dot(oh.reshape(BN3 * K, NBP), c_ref[...],
                 preferred_element_type=jnp.float32)
    s = (scores + cw).reshape(BN3, K, H)
    m = jnp.max(s, axis=1, keepdims=True)
    e = jnp.exp(s - m)
    w = e / jnp.sum(e, axis=1, keepdims=True)

    gt3 = tt.reshape(BN3, K, D)
    outs = []
    for hh in range(H):
        wh = w[:, :, hh]
        gth = gt3[:, :, hh * HD:(hh + 1) * HD]
        outs.append(jnp.sum(wh[:, :, None] * gth, axis=1))
    agg = jnp.concatenate(outs, axis=-1)

    res = agg + rw_ref[0, 0] * z_ref[...]
    out_ref[...] = jnp.maximum(res, 0.2 * res)


def _dense(z_p, u, gathered, bins_p, y_w, c_pad, rw):
    nrows = z_p.shape[0]
    return pl.pallas_call(
        _dense_body,
        grid=(nrows // BN3,),
        in_specs=[
            pl.BlockSpec((BN3, D), lambda i: (i, 0)),
            pl.BlockSpec((BN3, D), lambda i: (i, 0)),
            pl.BlockSpec((BN3 * K, D), lambda i: (i, 0)),
            pl.BlockSpec((BN3, K), lambda i: (i, 0)),
            pl.BlockSpec((H, D), lambda i: (0, 0)),
            pl.BlockSpec((NBP, H), lambda i: (0, 0)),
            pl.BlockSpec((1, 1), lambda i: (0, 0), memory_space=pltpu.SMEM),
        ],
        out_specs=pl.BlockSpec((BN3, D), lambda i: (i, 0)),
        out_shape=jax.ShapeDtypeStruct((nrows, D), jnp.float32),
    )(z_p, u, gathered, bins_p, y_w, c_pad, rw)


def kernel(z, A, neighbor_indices, affinity_bins, P_w, y_w, W_w, c_bins,
           residual_weight):
    n, d = z.shape
    pad = N_PAD - n
    z_p = jnp.pad(z, ((0, pad), (0, 0)))
    ni_p = jnp.pad(neighbor_indices, ((0, pad), (0, 0)))
    ab_p = jnp.pad(affinity_bins, ((0, pad), (0, 0)))
    c_pad = jnp.pad(c_bins, ((0, NBP - c_bins.shape[0]), (0, 0)))
    wcat = jnp.concatenate(
        [P_w[:, :D].T, P_w[:, D:].T, W_w.T], axis=1)

    u, vt = _proj(z_p, wcat)
    rw = residual_weight.reshape(1, 1)
    S = 2
    NS = N_PAD // S
    idx_flat = ni_p.reshape(-1)
    gathered = [_gather(idx_flat[s * NS * K:(s + 1) * NS * K], vt)
                for s in range(S)]
    outs = [
        _dense(z_p[s * NS:(s + 1) * NS], u[s * NS:(s + 1) * NS], gathered[s],
               ab_p[s * NS:(s + 1) * NS], y_w, c_pad, rw)
        for s in range(S)
    ]
    out_p = jnp.concatenate(outs, axis=0)
    return out_p[:n]


# speedup vs baseline: 3.2895x; 1.0310x over previous
import functools

import jax
import jax.numpy as jnp
from jax import lax
from jax.experimental import pallas as pl
from jax.experimental.pallas import tpu as pltpu
from jax.experimental.pallas import tpu_sc as plsc

N_PAD = 10240
K = 32
D = 128
H = 8
HD = D // H
VT = 2 * D
NBP = 16

NW = 32
NODES_PER_W = N_PAD // NW
GCHUNK = 4
EDGES_PER_STEP = GCHUNK * K

BN1 = 256
BN3 = 64


def _proj_body(z_ref, w_ref, u_ref, vt_ref):
    b = jnp.dot(z_ref[...], w_ref[...], preferred_element_type=jnp.float32)
    u_ref[...] = b[:, :D]
    vv = b[:, D:2 * D].astype(jnp.bfloat16)
    tt = b[:, 2 * D:].astype(jnp.bfloat16)
    lo = lax.bitcast_convert_type(vv, jnp.uint16).astype(jnp.uint32)
    hi = lax.bitcast_convert_type(tt, jnp.uint16).astype(jnp.uint32)
    vt_ref[...] = lax.bitcast_convert_type(lo | (hi << 16), jnp.int32)


def _proj(z_p, wcat):
    return pl.pallas_call(
        _proj_body,
        grid=(N_PAD // BN1,),
        in_specs=[
            pl.BlockSpec((BN1, D), lambda i: (i, 0)),
            pl.BlockSpec((D, 3 * D), lambda i: (0, 0)),
        ],
        out_specs=[
            pl.BlockSpec((BN1, D), lambda i: (i, 0)),
            pl.BlockSpec((BN1, D), lambda i: (i, 0)),
        ],
        out_shape=[
            jax.ShapeDtypeStruct((N_PAD, D), jnp.float32),
            jax.ShapeDtypeStruct((N_PAD, D), jnp.int32),
        ],
    )(z_p, wcat)


NBUF = 2


def _gather_body(idx_hbm, vt_hbm, out_hbm, ib, vt_sh, rows0, rows1,
                 g0, g1, w0, w1, *, nodes_per_w):
    EPS = EDGES_PER_STEP
    STEPS = nodes_per_w // GCHUNK
    wid = lax.axis_index("s") * 2 + lax.axis_index("c")
    ebase = wid * (nodes_per_w * K)
    bufs = (rows0, rows1)
    gsem = (g0, g1)
    wsem = (w0, w1)

    sid = lax.axis_index("s")
    SROWS = N_PAD // 16
    pltpu.sync_copy(vt_hbm.at[pl.ds(sid * SROWS, SROWS), :],
                    vt_sh.at[pl.ds(sid * SROWS, SROWS), :])
    plsc.subcore_barrier()

    pltpu.sync_copy(idx_hbm.at[pl.ds(ebase, nodes_per_w * K)], ib)

    def ib_at(ci):
        return ib.at[pl.ds(ci * EPS, EPS)]

    def out_at(ci):
        return out_hbm.at[pl.ds(ebase + ci * EPS, EPS), :]

    for b in range(NBUF):
        pltpu.async_copy(vt_sh.at[ib_at(b)], bufs[b], gsem[b])
    for b in range(NBUF):
        pltpu.make_async_copy(vt_sh.at[ib_at(b)], bufs[b], gsem[b]).wait()
        pltpu.async_copy(bufs[b], out_at(b), wsem[b])

    def rot(j, _):
        i0 = NBUF * j
        for b in range(NBUF):
            pltpu.make_async_copy(bufs[b], out_at(0), wsem[b]).wait()
            pltpu.async_copy(vt_sh.at[ib_at(i0 + b)], bufs[b], gsem[b])
        for b in range(NBUF):
            pltpu.make_async_copy(vt_sh.at[ib_at(i0 + b)], bufs[b],
                                  gsem[b]).wait()
            pltpu.async_copy(bufs[b], out_at(i0 + b), wsem[b])
        return 0

    lax.fori_loop(1, STEPS // NBUF, rot, 0)
    for b in range(NBUF):
        pltpu.make_async_copy(bufs[b], out_at(0), wsem[b]).wait()


@functools.cache
def _make_gather(nodes):
    nodes_per_w = nodes // NW
    return pl.kernel(
        functools.partial(_gather_body, nodes_per_w=nodes_per_w),
        mesh=plsc.VectorSubcoreMesh(core_axis_name="c", subcore_axis_name="s"),
        out_type=jax.ShapeDtypeStruct((nodes * K, D), jnp.int32),
        scratch_types=(
            [pltpu.VMEM((nodes_per_w * K,), jnp.int32),
             pltpu.VMEM_SHARED((N_PAD, D), jnp.int32)]
            + [pltpu.VMEM((EDGES_PER_STEP, D), jnp.int32)] * NBUF
            + [pltpu.SemaphoreType.DMA] * (2 * NBUF)
        ),
    )


def _gather(idx_flat, vt):
    return _make_gather(idx_flat.shape[0] // K)(idx_flat, vt)


def _dense_body(z_ref, u_ref, g_ref, bins_ref, y_ref, c_ref, rw_ref, out_ref):
    g = lax.bitcast_convert_type(g_ref[...], jnp.uint32)
    vv = lax.bitcast_convert_type(
        (g & 0xFFFF).astype(jnp.uint16), jnp.bfloat16).astype(jnp.float32)
    tt = lax.bitcast_convert_type(
        (g >> 16).astype(jnp.uint16), jnp.bfloat16).astype(jnp.float32)
    gv3 = vv.reshape(BN3, K, D)
    u3 = u_ref[...][:, None, :]
    h = u3 + gv3
    h = jnp.maximum(h, 0.2 * h)
    h2 = h.reshape(BN3 * K, D)
    scores = lax.dot_general(
        h2, y_ref[...],
        dimension_numbers=(((1,), (1,)), ((), ())),
        preferred_element_type=jnp.float32,
    )
    bins = bins_ref[...]
    ib = lax.broadcasted_iota(jnp.int32, (BN3, K, NBP), 2)
    oh = (ib == bins[:, :, None]).astype(jnp.float32)
    cw = jnp.dot(oh.reshape(BN3 * K, NBP), c_ref[...],
                 preferred_element_type=jnp.float32)
    s = (scores + cw).reshape(BN3, K, H)
    m = jnp.max(s, axis=1, keepdims=True)
    e = jnp.exp(s - m)
    w = e / jnp.sum(e, axis=1, keepdims=True)

    gt3 = tt.reshape(BN3, K, D)
    outs = []
    for hh in range(H):
        wh = w[:, :, hh]
        gth = gt3[:, :, hh * HD:(hh + 1) * HD]
        outs.append(jnp.sum(wh[:, :, None] * gth, axis=1))
    agg = jnp.concatenate(outs, axis=-1)

    res = agg + rw_ref[0, 0] * z_ref[...]
    out_ref[...] = jnp.maximum(res, 0.2 * res)


def _dense(z_p, u, gathered, bins_p, y_w, c_pad, rw):
    nrows = z_p.shape[0]
    return pl.pallas_call(
        _dense_body,
        grid=(nrows // BN3,),
        in_specs=[
            pl.BlockSpec((BN3, D), lambda i: (i, 0)),
            pl.BlockSpec((BN3, D), lambda i: (i, 0)),
            pl.BlockSpec((BN3 * K, D), lambda i: (i, 0)),
            pl.BlockSpec((BN3, K), lambda i: (i, 0)),
            pl.BlockSpec((H, D), lambda i: (0, 0)),
            pl.BlockSpec((NBP, H), lambda i: (0, 0)),
            pl.BlockSpec((1, 1), lambda i: (0, 0), memory_space=pltpu.SMEM),
        ],
        out_specs=pl.BlockSpec((BN3, D), lambda i: (i, 0)),
        out_shape=jax.ShapeDtypeStruct((nrows, D), jnp.float32),
    )(z_p, u, gathered, bins_p, y_w, c_pad, rw)


def kernel(z, A, neighbor_indices, affinity_bins, P_w, y_w, W_w, c_bins,
           residual_weight):
    n, d = z.shape
    pad = N_PAD - n
    z_p = jnp.pad(z, ((0, pad), (0, 0)))
    ni_p = jnp.pad(neighbor_indices, ((0, pad), (0, 0)))
    ab_p = jnp.pad(affinity_bins, ((0, pad), (0, 0)))
    c_pad = jnp.pad(c_bins, ((0, NBP - c_bins.shape[0]), (0, 0)))
    wcat = jnp.concatenate(
        [P_w[:, :D].T, P_w[:, D:].T, W_w.T], axis=1)

    u, vt = _proj(z_p, wcat)
    rw = residual_weight.reshape(1, 1)
    S = 4
    NS = N_PAD // S
    idx_flat = ni_p.reshape(-1)
    gathered = [_gather(idx_flat[s * NS * K:(s + 1) * NS * K], vt)
                for s in range(S)]
    outs = [
        _dense(z_p[s * NS:(s + 1) * NS], u[s * NS:(s + 1) * NS], gathered[s],
               ab_p[s * NS:(s + 1) * NS], y_w, c_pad, rw)
        for s in range(S)
    ]
    out_p = jnp.concatenate(outs, axis=0)
    return out_p[:n]
